# baseline (speedup 1.0000x reference)
import functools

import jax
import jax.numpy as jnp
from jax import lax
from jax.experimental import pallas as pl
from jax.experimental.pallas import tpu as pltpu
from jax.experimental.pallas import tpu_sc as plsc

N = 10000
E = 320000
D = 128
NC = 2
NS = 16
NW = NC * NS
NP = 10240
RPT = NP // NS
B = 96
NCH = 210
NQ = NCH // 3
WCH = 16
NWIN = 14
NCA = WCH * NWIN
TB = 1000
GRID = N // TB
DB = 128
DCA = 79
PRB = 64
PRC = NP // NW // PRB


def _sc_mesh():
    return plsc.VectorSubcoreMesh(
        core_axis_name="c", subcore_axis_name="s", num_cores=NC, num_subcores=NS
    )


def _prep_body(feat, perm_pad, dstd, deg_out, xp_out,
               pidx_v, prow_v, ones_v, didx_all, zrow_v, deg_sem, dacc_sh):
    c = lax.axis_index("c")
    s = lax.axis_index("s")
    w = c * NS + s

    def zb(i, carry):
        zrow_v[pl.ds(i * 16, 16)] = jnp.zeros((16,), jnp.float32)
        return carry
    lax.fori_loop(0, RPT // 16, zb, 0)
    pltpu.sync_copy(zrow_v, dacc_sh.at[pl.ds(s * RPT, RPT)])

    def ob(i, carry):
        ones_v[pl.ds(i * 16, 16)] = jnp.ones((16,), jnp.float32)
        return carry
    lax.fori_loop(0, DB // 16, ob, 0)
    plsc.subcore_barrier()

    pltpu.sync_copy(dstd.at[w], didx_all)

    def db(k, carry):
        pltpu.async_copy(ones_v, dacc_sh.at[didx_all.at[k]], deg_sem, add=True)
        return carry
    lax.fori_loop(0, DCA, db, 0)

    def gb(q, carry):
        base = w * (PRB * PRC) + q * PRB
        pltpu.sync_copy(perm_pad.at[pl.ds(base, PRB)], pidx_v)
        pltpu.sync_copy(feat.at[pidx_v], prow_v)
        pltpu.sync_copy(prow_v, xp_out.at[pl.ds(base, PRB)])
        return carry
    lax.fori_loop(0, PRC, gb, 0)

    def dw(k, carry):
        pltpu.make_async_copy(ones_v, dacc_sh.at[didx_all.at[k]],
                              deg_sem).wait()
        return carry
    lax.fori_loop(0, DCA, dw, 0)
    plsc.subcore_barrier()

    pltpu.sync_copy(dacc_sh.at[pl.ds(s * RPT, RPT)],
                    deg_out.at[c, pl.ds(s * RPT, RPT)])


_prep = pl.kernel(
    _prep_body,
    out_type=(jax.ShapeDtypeStruct((NC, NP), jnp.float32),
              jax.ShapeDtypeStruct((NP, D), jnp.float32)),
    mesh=_sc_mesh(),
    scratch_types=[
        pltpu.VMEM((PRB,), jnp.int32),
        pltpu.VMEM((PRB, D), jnp.float32),
        pltpu.VMEM((DB,), jnp.float32),
        pltpu.VMEM((DCA, DB), jnp.int32),
        pltpu.VMEM((RPT,), jnp.float32),
        pltpu.SemaphoreType.DMA,
        pltpu.VMEM_SHARED((NP,), jnp.float32),
    ],
)


def _agg_body(hs, src34, dst3, m_out,
              sidx, didx, rows_a, rows_b, rows_c,
              sa, sb, sc, iw, acc_sh):
    c = lax.axis_index("c")
    s = lax.axis_index("s")
    rows = (rows_a, rows_b, rows_c)
    sem = (sa, sb, sc)

    def zb(i, carry):
        def zc(j, carry2):
            rows_a[i, pl.ds(j * 16, 16)] = jnp.zeros((16,), jnp.float32)
            return carry2
        lax.fori_loop(0, D // 16, zc, 0)
        return carry
    lax.fori_loop(0, B, zb, 0)

    def zcopy(q, carry):
        pltpu.sync_copy(rows_a, acc_sh.at[pl.ds(s * RPT + q * B, B)])
        return carry
    lax.fori_loop(0, RPT // B, zcopy, 0)
    if RPT % B:
        pltpu.sync_copy(rows_a.at[pl.ds(0, RPT % B)],
                        acc_sh.at[pl.ds(s * RPT + (RPT // B) * B, RPT % B)])

    pltpu.sync_copy(src34.at[c, s, pl.ds(0, 2 * WCH)], sidx)
    pltpu.sync_copy(dst3.at[s, pl.ds(0, 2 * WCH)], didx)
    plsc.subcore_barrier()

    for j in range(3):
        pltpu.async_copy(hs.at[sidx.at[j]], rows[j], sem[j])

    def trip(p, carry):
        ka = 3 * p
        kwin = ka & 15
        w = ka >> 4
        in_body = (ka >= WCH) & (ka < WCH * (NWIN - 1))

        @pl.when((kwin < 3) & in_body)
        def _():
            roff = ((w + 1) & 1) * WCH
            pltpu.async_copy(src34.at[c, s, pl.ds((w + 1) * WCH, WCH)],
                             sidx.at[pl.ds(roff, WCH)], iw)
            pltpu.async_copy(dst3.at[s, pl.ds((w + 1) * WCH, WCH)],
                             didx.at[pl.ds(roff, WCH)], iw)

        for j in range(3):
            k = ka + j
            pltpu.make_async_copy(hs.at[sidx.at[k & 31]], rows[j],
                                  sem[j]).wait()
            pltpu.async_copy(rows[j], acc_sh.at[didx.at[k & 31]], sem[j],
                             add=True)

        @pl.when((kwin >= 8) & (kwin < 11) & in_body)
        def _():
            roff = ((w + 1) & 1) * WCH
            pltpu.make_async_copy(src34.at[c, s, pl.ds((w + 1) * WCH, WCH)],
                                  sidx.at[pl.ds(roff, WCH)], iw).wait()
            pltpu.make_async_copy(dst3.at[s, pl.ds((w + 1) * WCH, WCH)],
                                  didx.at[pl.ds(roff, WCH)], iw).wait()

        for j in range(3):
            k = ka + j
            pltpu.make_async_copy(rows[j], acc_sh.at[didx.at[k & 31]],
                                  sem[j]).wait()
            pltpu.async_copy(hs.at[sidx.at[(k + 3) & 31]], rows[j], sem[j])
        return carry
    lax.fori_loop(0, NQ, trip, 0)

    for j in range(3):
        pltpu.make_async_copy(hs.at[sidx.at[(NCH + j) & 31]], rows[j],
                              sem[j]).wait()
    plsc.subcore_barrier()

    pltpu.sync_copy(acc_sh.at[pl.ds(s * RPT, RPT)],
                    m_out.at[pl.ds(c * NP + s * RPT, RPT)])


_agg = pl.kernel(
    _agg_body,
    out_type=jax.ShapeDtypeStruct((2 * NP, D), jnp.float32),
    mesh=_sc_mesh(),
    scratch_types=(
        [pltpu.VMEM((2 * WCH, B), jnp.int32),
         pltpu.VMEM((2 * WCH, B), jnp.int32)]
        + [pltpu.VMEM((B, D), jnp.float32) for _ in range(3)]
        + [pltpu.SemaphoreType.DMA for _ in range(3)]
        + [pltpu.SemaphoreType.DMA,
           pltpu.VMEM_SHARED((NP, D), jnp.float32)]
    ),
)


def _t2_body(deg_ref, feat_ref, xp_ref, norm_ref, hs_ref):
    deg = deg_ref[0] + deg_ref[1]
    norm = jnp.where(deg > 0.0, lax.rsqrt(deg), 0.0)
    norm_ref[...] = norm
    hs_ref[0] = feat_ref[...] * norm
    hs_ref[1] = xp_ref[...] * norm


_t2 = pl.pallas_call(
    _t2_body,
    grid=(GRID,),
    in_specs=[
        pl.BlockSpec((2, TB, 1), lambda i: (0, i, 0)),
        pl.BlockSpec((TB, D), lambda i: (i, 0)),
        pl.BlockSpec((TB, D), lambda i: (i, 0)),
    ],
    out_specs=[
        pl.BlockSpec((TB, 1), lambda i: (i, 0)),
        pl.BlockSpec((2, TB, D), lambda i: (0, i, 0)),
    ],
    out_shape=[
        jax.ShapeDtypeStruct((NP, 1), jnp.float32),
        jax.ShapeDtypeStruct((2, NP, D), jnp.float32),
    ],
)


def _layer1_body(m_ref, norm_ref, w_ref, b_ref, out_ref):
    norm = norm_ref[...]
    w = w_ref[...]
    b = b_ref[...]
    for j in range(2):
        h = jnp.dot(m_ref[j] * norm, w, preferred_element_type=jnp.float32) + b
        h = jnp.maximum(h, 0.0)
        out_ref[j] = h * norm


_layer1 = pl.pallas_call(
    _layer1_body,
    grid=(GRID,),
    in_specs=[
        pl.BlockSpec((2, TB, D), lambda i: (0, i, 0)),
        pl.BlockSpec((TB, 1), lambda i: (i, 0)),
        pl.BlockSpec((D, D), lambda i: (0, 0)),
        pl.BlockSpec((1, D), lambda i: (0, 0)),
    ],
    out_specs=pl.BlockSpec((2, TB, D), lambda i: (0, i, 0)),
    out_shape=jax.ShapeDtypeStruct((2, NP, D), jnp.float32),
)


def _layer2_body(m_ref, norm_ref, w_ref, b_ref, out_ref, csum_ref):
    i = pl.program_id(0)
    norm = norm_ref[...]
    w = w_ref[...]
    b = b_ref[...]
    h0 = jnp.dot(m_ref[0] * norm, w, preferred_element_type=jnp.float32) + b
    h1 = jnp.dot(m_ref[1] * norm, w, preferred_element_type=jnp.float32) + b
    out_ref[0] = h0
    out_ref[1] = h1

    @pl.when(i == 0)
    def _():
        csum_ref[...] = jnp.zeros_like(csum_ref)

    csum_ref[...] += jnp.sum(h0, axis=0, keepdims=True)


_layer2 = pl.pallas_call(
    _layer2_body,
    grid=(GRID,),
    in_specs=[
        pl.BlockSpec((2, TB, D), lambda i: (0, i, 0)),
        pl.BlockSpec((TB, 1), lambda i: (i, 0)),
        pl.BlockSpec((D, D), lambda i: (0, 0)),
        pl.BlockSpec((1, D), lambda i: (0, 0)),
    ],
    out_specs=[
        pl.BlockSpec((2, TB, D), lambda i: (0, i, 0)),
        pl.BlockSpec((1, D), lambda i: (0, 0)),
    ],
    out_shape=[
        jax.ShapeDtypeStruct((2, NP, D), jnp.float32),
        jax.ShapeDtypeStruct((1, D), jnp.float32),
    ],
)


def _s_body(csum_ref, dw_ref, s_ref):
    summary = jax.nn.sigmoid(csum_ref[...] / N)
    s_ref[...] = lax.dot_general(
        summary, dw_ref[...], (((1,), (1,)), ((), ())),
        preferred_element_type=jnp.float32)


_s_kernel = pl.pallas_call(
    _s_body,
    out_shape=jax.ShapeDtypeStruct((1, D), jnp.float32),
)


def _loss_body(h2_ref, s_ref, loss_ref):
    i = pl.program_id(0)
    s = s_ref[...]
    pos = lax.dot_general(h2_ref[0], s, (((1,), (1,)), ((), ())),
                          preferred_element_type=jnp.float32)
    neg = lax.dot_general(h2_ref[1], s, (((1,), (1,)), ((), ())),
                          preferred_element_type=jnp.float32)

    def softplus(z):
        return jnp.maximum(z, 0.0) + jnp.log1p(jnp.exp(-jnp.abs(z)))

    tot = jnp.sum(softplus(-pos)) + jnp.sum(softplus(neg))

    @pl.when(i == 0)
    def _():
        loss_ref[...] = jnp.zeros_like(loss_ref)

    loss_ref[...] = loss_ref[...] + tot

    @pl.when(i == pl.num_programs(0) - 1)
    def _():
        loss_ref[...] = loss_ref[...] / N


_loss_kernel = pl.pallas_call(
    _loss_body,
    grid=(GRID,),
    in_specs=[
        pl.BlockSpec((2, TB, D), lambda i: (0, i, 0)),
        pl.BlockSpec((1, D), lambda i: (0, 0)),
    ],
    out_specs=pl.BlockSpec((1, 1), lambda i: (0, 0)),
    out_shape=jax.ShapeDtypeStruct((1, 1), jnp.float32),
)


@jax.jit
def kernel(features, edge_index, perm, W0, b0, W1, b1, disc_W):
    src = edge_index[0]
    dst = edge_index[1]
    perm_pad = jnp.concatenate([perm, jnp.zeros((NP - N,), jnp.int32)])

    ept = E // NS
    padw = NCA * B - ept
    src_r = src.reshape(NS, ept)
    dst_r = dst.reshape(NS, ept)
    src3 = jnp.concatenate(
        [src_r, jnp.zeros((NS, padw), jnp.int32)], axis=1).reshape(NS, NCA, B)
    dst3 = jnp.concatenate(
        [dst_r, jnp.full((NS, padw), NP - 1, jnp.int32)], axis=1
    ).reshape(NS, NCA, B)
    src34 = jnp.stack([src3, src3 + NP])

    epw = E // NW
    padd = DCA * DB - epw
    dstd = jnp.concatenate(
        [dst.reshape(NW, epw), jnp.full((NW, padd), NP - 1, jnp.int32)], axis=1
    ).reshape(NW, DCA, DB)

    deg2, xp = _prep(features, perm_pad, dstd)
    norm, hs = _t2(deg2.reshape(2, NP, 1), features, xp)
    m1 = _agg(hs.reshape(2 * NP, D), src34, dst3)
    h1s = _layer1(m1.reshape(2, NP, D), norm, W0, b0.reshape(1, D))
    m2 = _agg(h1s.reshape(2 * NP, D), src34, dst3)
    h2, csum = _layer2(m2.reshape(2, NP, D), norm, W1, b1.reshape(1, D))
    svec = _s_kernel(csum, disc_W)
    loss = _loss_kernel(h2, svec)
    return loss.reshape(())


# speedup vs baseline: 1.0114x; 1.0114x over previous
import functools

import jax
import jax.numpy as jnp
from jax import lax
from jax.experimental import pallas as pl
from jax.experimental.pallas import tpu as pltpu
from jax.experimental.pallas import tpu_sc as plsc

N = 10000
E = 320000
D = 128
NC = 2
NS = 16
NW = NC * NS
NP = 10240
RPT = NP // NS
B = 72
NCH = 280
NQ = NCH // 4
WCH = 16
NWIN = 18
NCA = WCH * NWIN
TB = 1000
GRID = N // TB
DB = 128
DCA = 79
PRB = 64
PRC = NP // NW // PRB


def _sc_mesh():
    return plsc.VectorSubcoreMesh(
        core_axis_name="c", subcore_axis_name="s", num_cores=NC, num_subcores=NS
    )


def _prep_body(feat, perm_pad, dstd, deg_out, xp_out,
               pidx_v, prow_v, ones_v, didx_all, zrow_v, deg_sem, dacc_sh):
    c = lax.axis_index("c")
    s = lax.axis_index("s")
    w = c * NS + s

    def zb(i, carry):
        zrow_v[pl.ds(i * 16, 16)] = jnp.zeros((16,), jnp.float32)
        return carry
    lax.fori_loop(0, RPT // 16, zb, 0)
    pltpu.sync_copy(zrow_v, dacc_sh.at[pl.ds(s * RPT, RPT)])

    def ob(i, carry):
        ones_v[pl.ds(i * 16, 16)] = jnp.ones((16,), jnp.float32)
        return carry
    lax.fori_loop(0, DB // 16, ob, 0)
    plsc.subcore_barrier()

    pltpu.sync_copy(dstd.at[w], didx_all)

    def db(k, carry):
        pltpu.async_copy(ones_v, dacc_sh.at[didx_all.at[k]], deg_sem, add=True)
        return carry
    lax.fori_loop(0, DCA, db, 0)

    def gb(q, carry):
        base = w * (PRB * PRC) + q * PRB
        pltpu.sync_copy(perm_pad.at[pl.ds(base, PRB)], pidx_v)
        pltpu.sync_copy(feat.at[pidx_v], prow_v)
        pltpu.sync_copy(prow_v, xp_out.at[pl.ds(base, PRB)])
        return carry
    lax.fori_loop(0, PRC, gb, 0)

    def dw(k, carry):
        pltpu.make_async_copy(ones_v, dacc_sh.at[didx_all.at[k]],
                              deg_sem).wait()
        return carry
    lax.fori_loop(0, DCA, dw, 0)
    plsc.subcore_barrier()

    pltpu.sync_copy(dacc_sh.at[pl.ds(s * RPT, RPT)],
                    deg_out.at[c, pl.ds(s * RPT, RPT)])


_prep = pl.kernel(
    _prep_body,
    out_type=(jax.ShapeDtypeStruct((NC, NP), jnp.float32),
              jax.ShapeDtypeStruct((NP, D), jnp.float32)),
    mesh=_sc_mesh(),
    scratch_types=[
        pltpu.VMEM((PRB,), jnp.int32),
        pltpu.VMEM((PRB, D), jnp.float32),
        pltpu.VMEM((DB,), jnp.float32),
        pltpu.VMEM((DCA, DB), jnp.int32),
        pltpu.VMEM((RPT,), jnp.float32),
        pltpu.SemaphoreType.DMA,
        pltpu.VMEM_SHARED((NP,), jnp.float32),
    ],
)


def _agg_body(hs, src34, dst3, m_out,
              sidx, didx, rows_a, rows_b, rows_c, rows_d,
              sa, sb, sc, sd, iw, acc_sh):
    c = lax.axis_index("c")
    s = lax.axis_index("s")
    rows = (rows_a, rows_b, rows_c, rows_d)
    sem = (sa, sb, sc, sd)

    def zb(i, carry):
        def zc(j, carry2):
            rows_a[i, pl.ds(j * 16, 16)] = jnp.zeros((16,), jnp.float32)
            return carry2
        lax.fori_loop(0, D // 16, zc, 0)
        return carry
    lax.fori_loop(0, B, zb, 0)

    def zcopy(q, carry):
        pltpu.sync_copy(rows_a, acc_sh.at[pl.ds(s * RPT + q * B, B)])
        return carry
    lax.fori_loop(0, RPT // B, zcopy, 0)
    if RPT % B:
        pltpu.sync_copy(rows_a.at[pl.ds(0, RPT % B)],
                        acc_sh.at[pl.ds(s * RPT + (RPT // B) * B, RPT % B)])

    pltpu.sync_copy(src34.at[c, s, pl.ds(0, 2 * WCH)], sidx)
    pltpu.sync_copy(dst3.at[s, pl.ds(0, 2 * WCH)], didx)
    plsc.subcore_barrier()

    for j in range(4):
        pltpu.async_copy(hs.at[sidx.at[j]], rows[j], sem[j])

    def trip(p, carry):
        ka = 4 * p
        kwin = ka & 15
        w = ka >> 4
        in_body = (ka >= WCH) & (ka < WCH * (NWIN - 1))

        @pl.when((kwin < 3) & in_body)
        def _():
            roff = ((w + 1) & 1) * WCH
            pltpu.async_copy(src34.at[c, s, pl.ds((w + 1) * WCH, WCH)],
                             sidx.at[pl.ds(roff, WCH)], iw)
            pltpu.async_copy(dst3.at[s, pl.ds((w + 1) * WCH, WCH)],
                             didx.at[pl.ds(roff, WCH)], iw)

        for j in range(4):
            k = ka + j
            pltpu.make_async_copy(hs.at[sidx.at[k & 31]], rows[j],
                                  sem[j]).wait()
            pltpu.async_copy(rows[j], acc_sh.at[didx.at[k & 31]], sem[j],
                             add=True)

        @pl.when((kwin >= 8) & (kwin < 11) & in_body)
        def _():
            roff = ((w + 1) & 1) * WCH
            pltpu.make_async_copy(src34.at[c, s, pl.ds((w + 1) * WCH, WCH)],
                                  sidx.at[pl.ds(roff, WCH)], iw).wait()
            pltpu.make_async_copy(dst3.at[s, pl.ds((w + 1) * WCH, WCH)],
                                  didx.at[pl.ds(roff, WCH)], iw).wait()

        for j in range(4):
            k = ka + j
            pltpu.make_async_copy(rows[j], acc_sh.at[didx.at[k & 31]],
                                  sem[j]).wait()
            pltpu.async_copy(hs.at[sidx.at[(k + 4) & 31]], rows[j], sem[j])
        return carry
    lax.fori_loop(0, NQ, trip, 0)

    for j in range(4):
        pltpu.make_async_copy(hs.at[sidx.at[(NCH + j) & 31]], rows[j],
                              sem[j]).wait()
    plsc.subcore_barrier()

    pltpu.sync_copy(acc_sh.at[pl.ds(s * RPT, RPT)],
                    m_out.at[pl.ds(c * NP + s * RPT, RPT)])


_agg = pl.kernel(
    _agg_body,
    out_type=jax.ShapeDtypeStruct((2 * NP, D), jnp.float32),
    mesh=_sc_mesh(),
    scratch_types=(
        [pltpu.VMEM((2 * WCH, B), jnp.int32),
         pltpu.VMEM((2 * WCH, B), jnp.int32)]
        + [pltpu.VMEM((B, D), jnp.float32) for _ in range(4)]
        + [pltpu.SemaphoreType.DMA for _ in range(4)]
        + [pltpu.SemaphoreType.DMA,
           pltpu.VMEM_SHARED((NP, D), jnp.float32)]
    ),
)


def _t2_body(deg_ref, feat_ref, xp_ref, norm_ref, hs_ref):
    deg = deg_ref[0] + deg_ref[1]
    norm = jnp.where(deg > 0.0, lax.rsqrt(deg), 0.0)
    norm_ref[...] = norm
    hs_ref[0] = feat_ref[...] * norm
    hs_ref[1] = xp_ref[...] * norm


_t2 = pl.pallas_call(
    _t2_body,
    grid=(GRID,),
    in_specs=[
        pl.BlockSpec((2, TB, 1), lambda i: (0, i, 0)),
        pl.BlockSpec((TB, D), lambda i: (i, 0)),
        pl.BlockSpec((TB, D), lambda i: (i, 0)),
    ],
    out_specs=[
        pl.BlockSpec((TB, 1), lambda i: (i, 0)),
        pl.BlockSpec((2, TB, D), lambda i: (0, i, 0)),
    ],
    out_shape=[
        jax.ShapeDtypeStruct((NP, 1), jnp.float32),
        jax.ShapeDtypeStruct((2, NP, D), jnp.float32),
    ],
)


def _layer1_body(m_ref, norm_ref, w_ref, b_ref, out_ref):
    norm = norm_ref[...]
    w = w_ref[...]
    b = b_ref[...]
    for j in range(2):
        h = jnp.dot(m_ref[j] * norm, w, preferred_element_type=jnp.float32) + b
        h = jnp.maximum(h, 0.0)
        out_ref[j] = h * norm


_layer1 = pl.pallas_call(
    _layer1_body,
    grid=(GRID,),
    in_specs=[
        pl.BlockSpec((2, TB, D), lambda i: (0, i, 0)),
        pl.BlockSpec((TB, 1), lambda i: (i, 0)),
        pl.BlockSpec((D, D), lambda i: (0, 0)),
        pl.BlockSpec((1, D), lambda i: (0, 0)),
    ],
    out_specs=pl.BlockSpec((2, TB, D), lambda i: (0, i, 0)),
    out_shape=jax.ShapeDtypeStruct((2, NP, D), jnp.float32),
)


def _layer2_body(m_ref, norm_ref, w_ref, b_ref, out_ref, csum_ref):
    i = pl.program_id(0)
    norm = norm_ref[...]
    w = w_ref[...]
    b = b_ref[...]
    h0 = jnp.dot(m_ref[0] * norm, w, preferred_element_type=jnp.float32) + b
    h1 = jnp.dot(m_ref[1] * norm, w, preferred_element_type=jnp.float32) + b
    out_ref[0] = h0
    out_ref[1] = h1

    @pl.when(i == 0)
    def _():
        csum_ref[...] = jnp.zeros_like(csum_ref)

    csum_ref[...] += jnp.sum(h0, axis=0, keepdims=True)


_layer2 = pl.pallas_call(
    _layer2_body,
    grid=(GRID,),
    in_specs=[
        pl.BlockSpec((2, TB, D), lambda i: (0, i, 0)),
        pl.BlockSpec((TB, 1), lambda i: (i, 0)),
        pl.BlockSpec((D, D), lambda i: (0, 0)),
        pl.BlockSpec((1, D), lambda i: (0, 0)),
    ],
    out_specs=[
        pl.BlockSpec((2, TB, D), lambda i: (0, i, 0)),
        pl.BlockSpec((1, D), lambda i: (0, 0)),
    ],
    out_shape=[
        jax.ShapeDtypeStruct((2, NP, D), jnp.float32),
        jax.ShapeDtypeStruct((1, D), jnp.float32),
    ],
)


def _s_body(csum_ref, dw_ref, s_ref):
    summary = jax.nn.sigmoid(csum_ref[...] / N)
    s_ref[...] = lax.dot_general(
        summary, dw_ref[...], (((1,), (1,)), ((), ())),
        preferred_element_type=jnp.float32)


_s_kernel = pl.pallas_call(
    _s_body,
    out_shape=jax.ShapeDtypeStruct((1, D), jnp.float32),
)


def _loss_body(h2_ref, s_ref, loss_ref):
    i = pl.program_id(0)
    s = s_ref[...]
    pos = lax.dot_general(h2_ref[0], s, (((1,), (1,)), ((), ())),
                          preferred_element_type=jnp.float32)
    neg = lax.dot_general(h2_ref[1], s, (((1,), (1,)), ((), ())),
                          preferred_element_type=jnp.float32)

    def softplus(z):
        return jnp.maximum(z, 0.0) + jnp.log1p(jnp.exp(-jnp.abs(z)))

    tot = jnp.sum(softplus(-pos)) + jnp.sum(softplus(neg))

    @pl.when(i == 0)
    def _():
        loss_ref[...] = jnp.zeros_like(loss_ref)

    loss_ref[...] = loss_ref[...] + tot

    @pl.when(i == pl.num_programs(0) - 1)
    def _():
        loss_ref[...] = loss_ref[...] / N


_loss_kernel = pl.pallas_call(
    _loss_body,
    grid=(GRID,),
    in_specs=[
        pl.BlockSpec((2, TB, D), lambda i: (0, i, 0)),
        pl.BlockSpec((1, D), lambda i: (0, 0)),
    ],
    out_specs=pl.BlockSpec((1, 1), lambda i: (0, 0)),
    out_shape=jax.ShapeDtypeStruct((1, 1), jnp.float32),
)


@jax.jit
def kernel(features, edge_index, perm, W0, b0, W1, b1, disc_W):
    src = edge_index[0]
    dst = edge_index[1]
    perm_pad = jnp.concatenate([perm, jnp.zeros((NP - N,), jnp.int32)])

    ept = E // NS
    padw = NCA * B - ept
    src_r = src.reshape(NS, ept)
    dst_r = dst.reshape(NS, ept)
    src3 = jnp.concatenate(
        [src_r, jnp.zeros((NS, padw), jnp.int32)], axis=1).reshape(NS, NCA, B)
    dst3 = jnp.concatenate(
        [dst_r, jnp.full((NS, padw), NP - 1, jnp.int32)], axis=1
    ).reshape(NS, NCA, B)
    src34 = jnp.stack([src3, src3 + NP])

    epw = E // NW
    padd = DCA * DB - epw
    dstd = jnp.concatenate(
        [dst.reshape(NW, epw), jnp.full((NW, padd), NP - 1, jnp.int32)], axis=1
    ).reshape(NW, DCA, DB)

    deg2, xp = _prep(features, perm_pad, dstd)
    norm, hs = _t2(deg2.reshape(2, NP, 1), features, xp)
    m1 = _agg(hs.reshape(2 * NP, D), src34, dst3)
    h1s = _layer1(m1.reshape(2, NP, D), norm, W0, b0.reshape(1, D))
    m2 = _agg(h1s.reshape(2 * NP, D), src34, dst3)
    h2, csum = _layer2(m2.reshape(2, NP, D), norm, W1, b1.reshape(1, D))
    svec = _s_kernel(csum, disc_W)
    loss = _loss_kernel(h2, svec)
    return loss.reshape(())


# speedup vs baseline: 1.0475x; 1.0357x over previous
import functools

import jax
import jax.numpy as jnp
from jax import lax
from jax.experimental import pallas as pl
from jax.experimental.pallas import tpu as pltpu
from jax.experimental.pallas import tpu_sc as plsc

N = 10000
E = 320000
D = 128
NC = 2
NS = 16
NW = NC * NS
NP = 10240
RPT = NP // NS
B = 80
NCH = 252
NQ = NCH // 3
WCH = 32
NWIN = 8
NCA = WCH * NWIN
TB = 1000
GRID = N // TB
DB = 128
DCA = 79
PRB = 64
PRC = NP // NW // PRB


def _sc_mesh():
    return plsc.VectorSubcoreMesh(
        core_axis_name="c", subcore_axis_name="s", num_cores=NC, num_subcores=NS
    )


def _prep_body(feat, perm_pad, dstd, deg_out, xp_out,
               pidx_v, prow_v, ones_v, didx_all, zrow_v, deg_sem, dacc_sh):
    c = lax.axis_index("c")
    s = lax.axis_index("s")
    w = c * NS + s

    def zb(i, carry):
        zrow_v[pl.ds(i * 16, 16)] = jnp.zeros((16,), jnp.float32)
        return carry
    lax.fori_loop(0, RPT // 16, zb, 0)
    pltpu.sync_copy(zrow_v, dacc_sh.at[pl.ds(s * RPT, RPT)])

    def ob(i, carry):
        ones_v[pl.ds(i * 16, 16)] = jnp.ones((16,), jnp.float32)
        return carry
    lax.fori_loop(0, DB // 16, ob, 0)
    plsc.subcore_barrier()

    pltpu.sync_copy(dstd.at[w], didx_all)

    def db(k, carry):
        pltpu.async_copy(ones_v, dacc_sh.at[didx_all.at[k]], deg_sem, add=True)
        return carry
    lax.fori_loop(0, DCA, db, 0)

    def gb(q, carry):
        base = w * (PRB * PRC) + q * PRB
        pltpu.sync_copy(perm_pad.at[pl.ds(base, PRB)], pidx_v)
        pltpu.sync_copy(feat.at[pidx_v], prow_v)
        pltpu.sync_copy(prow_v, xp_out.at[pl.ds(base, PRB)])
        return carry
    lax.fori_loop(0, PRC, gb, 0)

    def dw(k, carry):
        pltpu.make_async_copy(ones_v, dacc_sh.at[didx_all.at[k]],
                              deg_sem).wait()
        return carry
    lax.fori_loop(0, DCA, dw, 0)
    plsc.subcore_barrier()

    pltpu.sync_copy(dacc_sh.at[pl.ds(s * RPT, RPT)],
                    deg_out.at[c, pl.ds(s * RPT, RPT)])


_prep = pl.kernel(
    _prep_body,
    out_type=(jax.ShapeDtypeStruct((NC, NP), jnp.float32),
              jax.ShapeDtypeStruct((NP, D), jnp.float32)),
    mesh=_sc_mesh(),
    scratch_types=[
        pltpu.VMEM((PRB,), jnp.int32),
        pltpu.VMEM((PRB, D), jnp.float32),
        pltpu.VMEM((DB,), jnp.float32),
        pltpu.VMEM((DCA, DB), jnp.int32),
        pltpu.VMEM((RPT,), jnp.float32),
        pltpu.SemaphoreType.DMA,
        pltpu.VMEM_SHARED((NP,), jnp.float32),
    ],
)


def _agg_body(hs, src34, dst3, m_out,
              sidx, didx, rows_a, rows_b, rows_c,
              sa, sb, sc, iw, acc_sh):
    c = lax.axis_index("c")
    s = lax.axis_index("s")
    rows = (rows_a, rows_b, rows_c)
    sem = (sa, sb, sc)

    def zb(i, carry):
        def zc(j, carry2):
            rows_a[i, pl.ds(j * 16, 16)] = jnp.zeros((16,), jnp.float32)
            return carry2
        lax.fori_loop(0, D // 16, zc, 0)
        return carry
    lax.fori_loop(0, B, zb, 0)

    def zcopy(q, carry):
        pltpu.sync_copy(rows_a, acc_sh.at[pl.ds(s * RPT + q * B, B)])
        return carry
    lax.fori_loop(0, RPT // B, zcopy, 0)
    if RPT % B:
        pltpu.sync_copy(rows_a.at[pl.ds(0, RPT % B)],
                        acc_sh.at[pl.ds(s * RPT + (RPT // B) * B, RPT % B)])

    pltpu.sync_copy(src34.at[c, s, pl.ds(0, 2 * WCH)], sidx)
    pltpu.sync_copy(dst3.at[s, pl.ds(0, 2 * WCH)], didx)
    plsc.subcore_barrier()

    for j in range(3):
        pltpu.async_copy(hs.at[sidx.at[j]], rows[j], sem[j])

    def trip(p, carry):
        ka = 3 * p
        kwin = ka & 31
        w = ka >> 5
        in_body = (ka >= WCH) & (ka < WCH * (NWIN - 1))

        @pl.when((kwin >= 3) & (kwin < 6) & in_body)
        def _():
            roff = ((w + 1) & 1) * WCH
            pltpu.async_copy(src34.at[c, s, pl.ds((w + 1) * WCH, WCH)],
                             sidx.at[pl.ds(roff, WCH)], iw)
            pltpu.async_copy(dst3.at[s, pl.ds((w + 1) * WCH, WCH)],
                             didx.at[pl.ds(roff, WCH)], iw)

        for j in range(3):
            k = ka + j
            pltpu.make_async_copy(hs.at[sidx.at[k & 63]], rows[j],
                                  sem[j]).wait()
            pltpu.async_copy(rows[j], acc_sh.at[didx.at[k & 63]], sem[j],
                             add=True)

        @pl.when((kwin >= 24) & (kwin < 27) & in_body)
        def _():
            roff = ((w + 1) & 1) * WCH
            pltpu.make_async_copy(src34.at[c, s, pl.ds((w + 1) * WCH, WCH)],
                                  sidx.at[pl.ds(roff, WCH)], iw).wait()
            pltpu.make_async_copy(dst3.at[s, pl.ds((w + 1) * WCH, WCH)],
                                  didx.at[pl.ds(roff, WCH)], iw).wait()

        for j in range(3):
            k = ka + j
            pltpu.make_async_copy(rows[j], acc_sh.at[didx.at[k & 63]],
                                  sem[j]).wait()
            pltpu.async_copy(hs.at[sidx.at[(k + 3) & 63]], rows[j], sem[j])
        return carry
    lax.fori_loop(0, NQ, trip, 0)

    for j in range(3):
        pltpu.make_async_copy(hs.at[sidx.at[(NCH + j) & 63]], rows[j],
                              sem[j]).wait()
    plsc.subcore_barrier()

    pltpu.sync_copy(acc_sh.at[pl.ds(s * RPT, RPT)],
                    m_out.at[pl.ds(c * NP + s * RPT, RPT)])


_agg = pl.kernel(
    _agg_body,
    out_type=jax.ShapeDtypeStruct((2 * NP, D), jnp.float32),
    mesh=_sc_mesh(),
    scratch_types=(
        [pltpu.VMEM((2 * WCH, B), jnp.int32),
         pltpu.VMEM((2 * WCH, B), jnp.int32)]
        + [pltpu.VMEM((B, D), jnp.float32) for _ in range(3)]
        + [pltpu.SemaphoreType.DMA for _ in range(3)]
        + [pltpu.SemaphoreType.DMA,
           pltpu.VMEM_SHARED((NP, D), jnp.float32)]
    ),
)


def _t2_body(deg_ref, feat_ref, xp_ref, norm_ref, hs_ref):
    deg = deg_ref[0] + deg_ref[1]
    norm = jnp.where(deg > 0.0, lax.rsqrt(deg), 0.0)
    norm_ref[...] = norm
    hs_ref[0] = feat_ref[...] * norm
    hs_ref[1] = xp_ref[...] * norm


_t2 = pl.pallas_call(
    _t2_body,
    grid=(GRID,),
    in_specs=[
        pl.BlockSpec((2, TB, 1), lambda i: (0, i, 0)),
        pl.BlockSpec((TB, D), lambda i: (i, 0)),
        pl.BlockSpec((TB, D), lambda i: (i, 0)),
    ],
    out_specs=[
        pl.BlockSpec((TB, 1), lambda i: (i, 0)),
        pl.BlockSpec((2, TB, D), lambda i: (0, i, 0)),
    ],
    out_shape=[
        jax.ShapeDtypeStruct((NP, 1), jnp.float32),
        jax.ShapeDtypeStruct((2, NP, D), jnp.float32),
    ],
)


def _layer1_body(m_ref, norm_ref, w_ref, b_ref, out_ref):
    norm = norm_ref[...]
    w = w_ref[...]
    b = b_ref[...]
    for j in range(2):
        h = jnp.dot(m_ref[j] * norm, w, preferred_element_type=jnp.float32) + b
        h = jnp.maximum(h, 0.0)
        out_ref[j] = h * norm


_layer1 = pl.pallas_call(
    _layer1_body,
    grid=(GRID,),
    in_specs=[
        pl.BlockSpec((2, TB, D), lambda i: (0, i, 0)),
        pl.BlockSpec((TB, 1), lambda i: (i, 0)),
        pl.BlockSpec((D, D), lambda i: (0, 0)),
        pl.BlockSpec((1, D), lambda i: (0, 0)),
    ],
    out_specs=pl.BlockSpec((2, TB, D), lambda i: (0, i, 0)),
    out_shape=jax.ShapeDtypeStruct((2, NP, D), jnp.float32),
)


def _layer2_body(m_ref, norm_ref, w_ref, b_ref, out_ref, csum_ref):
    i = pl.program_id(0)
    norm = norm_ref[...]
    w = w_ref[...]
    b = b_ref[...]
    h0 = jnp.dot(m_ref[0] * norm, w, preferred_element_type=jnp.float32) + b
    h1 = jnp.dot(m_ref[1] * norm, w, preferred_element_type=jnp.float32) + b
    out_ref[0] = h0
    out_ref[1] = h1

    @pl.when(i == 0)
    def _():
        csum_ref[...] = jnp.zeros_like(csum_ref)

    csum_ref[...] += jnp.sum(h0, axis=0, keepdims=True)


_layer2 = pl.pallas_call(
    _layer2_body,
    grid=(GRID,),
    in_specs=[
        pl.BlockSpec((2, TB, D), lambda i: (0, i, 0)),
        pl.BlockSpec((TB, 1), lambda i: (i, 0)),
        pl.BlockSpec((D, D), lambda i: (0, 0)),
        pl.BlockSpec((1, D), lambda i: (0, 0)),
    ],
    out_specs=[
        pl.BlockSpec((2, TB, D), lambda i: (0, i, 0)),
        pl.BlockSpec((1, D), lambda i: (0, 0)),
    ],
    out_shape=[
        jax.ShapeDtypeStruct((2, NP, D), jnp.float32),
        jax.ShapeDtypeStruct((1, D), jnp.float32),
    ],
)


def _s_body(csum_ref, dw_ref, s_ref):
    summary = jax.nn.sigmoid(csum_ref[...] / N)
    s_ref[...] = lax.dot_general(
        summary, dw_ref[...], (((1,), (1,)), ((), ())),
        preferred_element_type=jnp.float32)


_s_kernel = pl.pallas_call(
    _s_body,
    out_shape=jax.ShapeDtypeStruct((1, D), jnp.float32),
)


def _loss_body(h2_ref, s_ref, loss_ref):
    i = pl.program_id(0)
    s = s_ref[...]
    pos = lax.dot_general(h2_ref[0], s, (((1,), (1,)), ((), ())),
                          preferred_element_type=jnp.float32)
    neg = lax.dot_general(h2_ref[1], s, (((1,), (1,)), ((), ())),
                          preferred_element_type=jnp.float32)

    def softplus(z):
        return jnp.maximum(z, 0.0) + jnp.log1p(jnp.exp(-jnp.abs(z)))

    tot = jnp.sum(softplus(-pos)) + jnp.sum(softplus(neg))

    @pl.when(i == 0)
    def _():
        loss_ref[...] = jnp.zeros_like(loss_ref)

    loss_ref[...] = loss_ref[...] + tot

    @pl.when(i == pl.num_programs(0) - 1)
    def _():
        loss_ref[...] = loss_ref[...] / N


_loss_kernel = pl.pallas_call(
    _loss_body,
    grid=(GRID,),
    in_specs=[
        pl.BlockSpec((2, TB, D), lambda i: (0, i, 0)),
        pl.BlockSpec((1, D), lambda i: (0, 0)),
    ],
    out_specs=pl.BlockSpec((1, 1), lambda i: (0, 0)),
    out_shape=jax.ShapeDtypeStruct((1, 1), jnp.float32),
)


@jax.jit
def kernel(features, edge_index, perm, W0, b0, W1, b1, disc_W):
    src = edge_index[0]
    dst = edge_index[1]
    perm_pad = jnp.concatenate([perm, jnp.zeros((NP - N,), jnp.int32)])

    ept = E // NS
    padw = NCA * B - ept
    src_r = src.reshape(NS, ept)
    dst_r = dst.reshape(NS, ept)
    src3 = jnp.concatenate(
        [src_r, jnp.zeros((NS, padw), jnp.int32)], axis=1).reshape(NS, NCA, B)
    dst3 = jnp.concatenate(
        [dst_r, jnp.full((NS, padw), NP - 1, jnp.int32)], axis=1
    ).reshape(NS, NCA, B)
    src34 = jnp.stack([src3, src3 + NP])

    epw = E // NW
    padd = DCA * DB - epw
    dstd = jnp.concatenate(
        [dst.reshape(NW, epw), jnp.full((NW, padd), NP - 1, jnp.int32)], axis=1
    ).reshape(NW, DCA, DB)

    deg2, xp = _prep(features, perm_pad, dstd)
    norm, hs = _t2(deg2.reshape(2, NP, 1), features, xp)
    m1 = _agg(hs.reshape(2 * NP, D), src34, dst3)
    h1s = _layer1(m1.reshape(2, NP, D), norm, W0, b0.reshape(1, D))
    m2 = _agg(h1s.reshape(2 * NP, D), src34, dst3)
    h2, csum = _layer2(m2.reshape(2, NP, D), norm, W1, b1.reshape(1, D))
    svec = _s_kernel(csum, disc_W)
    loss = _loss_kernel(h2, svec)
    return loss.reshape(())
